# baseline (speedup 1.0000x reference)
import jax
import jax.numpy as jnp
from jax.experimental import pallas as pl
from jax.experimental.pallas import tpu as pltpu

B, C, H, W = 8, 3, 1024, 1024
CLAMP_MIN = 1e-8
BH = 512
CHUNK = 16
LOG2E = 1.4426950408889634
LN2 = 0.6931471805599453


def _ce_kernel(x_ref, y_ref, w_ref, m_ref, out_ref):
    log2e = jnp.float32(LOG2E)
    w0 = w_ref[0, 0]
    w1 = w_ref[0, 1]
    w2 = w_ref[0, 2]

    def body(i, acc):
        r = pl.ds(i * CHUNK, CHUNK)
        b0 = x_ref[0, 0, r, :] * log2e
        b1 = x_ref[0, 1, r, :] * log2e
        b2 = x_ref[0, 2, r, :] * log2e
        mb = jnp.maximum(jnp.maximum(b0, b1), b2)
        c0 = b0 - mb
        c1 = b1 - mb
        c2 = b2 - mb
        s = jnp.exp2(c0) + jnp.exp2(c1) + jnp.exp2(c2)

        y = y_ref[0, r, :]
        m1 = y == 1
        m2 = y == 2
        c_y = jnp.where(m1, c1, jnp.where(m2, c2, c0))
        logp2_y = jnp.maximum(c_y - jnp.log2(s), jnp.float32(jnp.log2(CLAMP_MIN)))
        w_y = jnp.where(m1, w1, jnp.where(m2, w2, w0))
        return acc + w_y * logp2_y * m_ref[0, r, :]

    acc = jnp.zeros((CHUNK, W), dtype=jnp.float32)
    acc = jax.lax.fori_loop(0, BH // CHUNK, body, acc, unroll=2)
    out_ref[0, 0, :, :] = jnp.sum(acc).reshape(1, 1)


def kernel(x, y, weight, loss_mask):
    grid = (B, H // BH)
    partials = pl.pallas_call(
        _ce_kernel,
        grid=grid,
        in_specs=[
            pl.BlockSpec((1, C, BH, W), lambda i, j: (i, 0, j, 0)),
            pl.BlockSpec((1, BH, W), lambda i, j: (i, j, 0)),
            pl.BlockSpec((1, C), lambda i, j: (0, 0)),
            pl.BlockSpec((1, BH, W), lambda i, j: (i, j, 0)),
        ],
        out_specs=pl.BlockSpec((1, 1, 1, 1), lambda i, j: (i, j, 0, 0)),
        out_shape=jax.ShapeDtypeStruct(grid + (1, 1), jnp.float32),
        compiler_params=pltpu.CompilerParams(
            dimension_semantics=("parallel", "parallel"),
        ),
    )(x, y, weight.reshape(1, C), loss_mask)
    scale = jnp.float32(-LN2 / (B * H * W))
    return jnp.sum(partials) * scale


# speedup vs baseline: 1.0394x; 1.0394x over previous
import jax
import jax.numpy as jnp
from jax.experimental import pallas as pl
from jax.experimental.pallas import tpu as pltpu

B, C, H, W = 8, 3, 1024, 1024
CLAMP_MIN = 1e-8
BH = 1024
CHUNK = 16
LOG2E = 1.4426950408889634
LN2 = 0.6931471805599453


def _ce_kernel(x_ref, y_ref, w_ref, m_ref, out_ref):
    log2e = jnp.float32(LOG2E)
    w0 = w_ref[0, 0]
    w1 = w_ref[0, 1]
    w2 = w_ref[0, 2]

    def body(i, acc):
        r = pl.ds(i * CHUNK, CHUNK)
        b0 = x_ref[0, 0, r, :] * log2e
        b1 = x_ref[0, 1, r, :] * log2e
        b2 = x_ref[0, 2, r, :] * log2e
        mb = jnp.maximum(jnp.maximum(b0, b1), b2)
        c0 = b0 - mb
        c1 = b1 - mb
        c2 = b2 - mb
        s = jnp.exp2(c0) + jnp.exp2(c1) + jnp.exp2(c2)

        y = y_ref[0, r, :]
        m1 = y == 1
        m2 = y == 2
        c_y = jnp.where(m1, c1, jnp.where(m2, c2, c0))
        logp2_y = jnp.maximum(c_y - jnp.log2(s), jnp.float32(jnp.log2(CLAMP_MIN)))
        w_y = jnp.where(m1, w1, jnp.where(m2, w2, w0))
        return acc + w_y * logp2_y * m_ref[0, r, :]

    acc = jnp.zeros((CHUNK, W), dtype=jnp.float32)
    acc = jax.lax.fori_loop(0, BH // CHUNK, body, acc, unroll=2)
    out_ref[0, 0, :, :] = jnp.sum(acc).reshape(1, 1)


def kernel(x, y, weight, loss_mask):
    grid = (B * (H // BH),)
    partials = pl.pallas_call(
        _ce_kernel,
        grid=grid,
        in_specs=[
            pl.BlockSpec((1, C, BH, W), lambda i: (i, 0, 0, 0)),
            pl.BlockSpec((1, BH, W), lambda i: (i, 0, 0)),
            pl.BlockSpec((1, C), lambda i: (0, 0)),
            pl.BlockSpec((1, BH, W), lambda i: (i, 0, 0)),
        ],
        out_specs=pl.BlockSpec((1, 1, 1, 1), lambda i: (i, 0, 0, 0)),
        out_shape=jax.ShapeDtypeStruct(grid + (1, 1, 1), jnp.float32),
        compiler_params=pltpu.CompilerParams(
            dimension_semantics=("parallel",),
        ),
    )(x, y, weight.reshape(1, C), loss_mask)
    scale = jnp.float32(-LN2 / (B * H * W))
    return jnp.sum(partials) * scale
